# Initial kernel scaffold; baseline (speedup 1.0000x reference)
import functools

import jax
import jax.numpy as jnp
from jax import lax
from jax.experimental import pallas as pl
from jax.experimental.pallas import tpu as pltpu
from jax.experimental.pallas import tpu_sc as plsc

N = 10000
E = 320000
D_IN = 128
HID = 128
D_OUT = 128
NUM_RELS = 8
NUM_BASES = 4

_NC = 2
_NS = 16
_NW = _NC * _NS
_ET = E // _NW
_CH = 80
_NCHUNK = _ET // _CH
_RPT = N // _NS

_BLK = 400
_NBLK = N // _BLK


def _prep_body(coeff0_ref, basis0_ref, coeff1_ref, basis1_ref, et_ref, src_ref,
               w0_ref, w1_ref, gidx_ref):
    w0_ref[...] = jnp.dot(coeff0_ref[...], basis0_ref[...],
                          preferred_element_type=jnp.float32)
    w1_ref[...] = jnp.dot(coeff1_ref[...], basis1_ref[...],
                          preferred_element_type=jnp.float32)
    gidx_ref[...] = et_ref[...] * N + src_ref[...]


def _prep(coeff0, basis0_2d, coeff1, basis1_2d, et2d, src2d):
    return pl.pallas_call(
        _prep_body,
        out_shape=(
            jax.ShapeDtypeStruct((NUM_RELS, HID * HID), jnp.float32),
            jax.ShapeDtypeStruct((NUM_RELS, HID * HID), jnp.float32),
            jax.ShapeDtypeStruct((E // 128, 128), jnp.int32),
        ),
    )(coeff0, basis0_2d, coeff1, basis1_2d, et2d, src2d)


def _proj_first_body(x_ref, win_ref, bin_ref, w_ref, wself_ref, bias_ref,
                     hproj_ref, selfp_ref):
    h = jnp.maximum(
        jnp.dot(x_ref[...], win_ref[...], preferred_element_type=jnp.float32)
        + bin_ref[...], 0.0)
    selfp_ref[...] = jnp.dot(h, wself_ref[...],
                             preferred_element_type=jnp.float32) + bias_ref[...]
    for r in range(NUM_RELS):
        hproj_ref[r] = jnp.dot(h, w_ref[r], preferred_element_type=jnp.float32)


def _proj_mid_body(agg_ref, selfp_prev_ref, w_ref, wself_ref, bias_ref,
                   hproj_ref, selfp_ref):
    h = jnp.maximum(agg_ref[0] + agg_ref[1] + selfp_prev_ref[...], 0.0)
    selfp_ref[...] = jnp.dot(h, wself_ref[...],
                             preferred_element_type=jnp.float32) + bias_ref[...]
    for r in range(NUM_RELS):
        hproj_ref[r] = jnp.dot(h, w_ref[r], preferred_element_type=jnp.float32)


def _out_body(agg_ref, selfp_prev_ref, wout_ref, bout_ref, out_ref):
    h = jnp.maximum(agg_ref[0] + agg_ref[1] + selfp_prev_ref[...], 0.0)
    out_ref[...] = jnp.maximum(
        jnp.dot(h, wout_ref[...], preferred_element_type=jnp.float32)
        + bout_ref[...], 0.0)


def _proj_first(x, W_in, b_in, W, Wself, bias):
    return pl.pallas_call(
        _proj_first_body,
        grid=(_NBLK,),
        in_specs=[
            pl.BlockSpec((_BLK, D_IN), lambda i: (i, 0)),
            pl.BlockSpec((D_IN, HID), lambda i: (0, 0)),
            pl.BlockSpec((1, HID), lambda i: (0, 0)),
            pl.BlockSpec((NUM_RELS, HID, HID), lambda i: (0, 0, 0)),
            pl.BlockSpec((HID, HID), lambda i: (0, 0)),
            pl.BlockSpec((1, HID), lambda i: (0, 0)),
        ],
        out_specs=[
            pl.BlockSpec((NUM_RELS, _BLK, HID), lambda i: (0, i, 0)),
            pl.BlockSpec((_BLK, HID), lambda i: (i, 0)),
        ],
        out_shape=(
            jax.ShapeDtypeStruct((NUM_RELS, N, HID), jnp.float32),
            jax.ShapeDtypeStruct((N, HID), jnp.float32),
        ),
    )(x, W_in, b_in, W, Wself, bias)


def _proj_mid(aggp, selfp_prev, W, Wself, bias):
    return pl.pallas_call(
        _proj_mid_body,
        grid=(_NBLK,),
        in_specs=[
            pl.BlockSpec((_NC, _BLK, HID), lambda i: (0, i, 0)),
            pl.BlockSpec((_BLK, HID), lambda i: (i, 0)),
            pl.BlockSpec((NUM_RELS, HID, HID), lambda i: (0, 0, 0)),
            pl.BlockSpec((HID, HID), lambda i: (0, 0)),
            pl.BlockSpec((1, HID), lambda i: (0, 0)),
        ],
        out_specs=[
            pl.BlockSpec((NUM_RELS, _BLK, HID), lambda i: (0, i, 0)),
            pl.BlockSpec((_BLK, HID), lambda i: (i, 0)),
        ],
        out_shape=(
            jax.ShapeDtypeStruct((NUM_RELS, N, HID), jnp.float32),
            jax.ShapeDtypeStruct((N, HID), jnp.float32),
        ),
    )(aggp, selfp_prev, W, Wself, bias)


def _out_proj(aggp, selfp_prev, W_out, b_out):
    return pl.pallas_call(
        _out_body,
        grid=(_NBLK,),
        in_specs=[
            pl.BlockSpec((_NC, _BLK, HID), lambda i: (0, i, 0)),
            pl.BlockSpec((_BLK, HID), lambda i: (i, 0)),
            pl.BlockSpec((HID, D_OUT), lambda i: (0, 0)),
            pl.BlockSpec((1, D_OUT), lambda i: (0, 0)),
        ],
        out_specs=pl.BlockSpec((_BLK, D_OUT), lambda i: (i, 0)),
        out_shape=jax.ShapeDtypeStruct((N, D_OUT), jnp.float32),
    )(aggp, selfp_prev, W_out, b_out)


_sc_mesh = plsc.VectorSubcoreMesh(core_axis_name="c", subcore_axis_name="s")


@functools.partial(
    pl.kernel,
    out_type=jax.ShapeDtypeStruct((_NC, N, HID), jnp.float32),
    mesh=_sc_mesh,
    scratch_types=[
        pltpu.VMEM((_NCHUNK, _CH), jnp.int32),
        pltpu.VMEM((_NCHUNK, _CH), jnp.int32),
        pltpu.VMEM((_CH, HID), jnp.float32),
        pltpu.VMEM((_CH, HID), jnp.float32),
        pltpu.VMEM_SHARED((N, HID), jnp.float32),
        pltpu.SemaphoreType.DMA,
        pltpu.SemaphoreType.DMA,
    ],
)
def _sc_edge_agg(hproj_hbm, gidx_hbm, dst_hbm, zeros_hbm, out_hbm,
                 gidx_v, dst_v, rows_a, rows_b, acc_sh, sem_a, sem_b):
    c = lax.axis_index("c")
    s = lax.axis_index("s")
    wid = c * _NS + s

    pltpu.sync_copy(gidx_hbm.at[pl.ds(wid * _NCHUNK, _NCHUNK)], gidx_v)
    pltpu.sync_copy(dst_hbm.at[pl.ds(wid * _NCHUNK, _NCHUNK)], dst_v)

    pltpu.sync_copy(zeros_hbm, acc_sh.at[pl.ds(s * _RPT, _RPT)])
    plsc.subcore_barrier()

    pltpu.async_copy(hproj_hbm.at[gidx_v.at[0]], rows_a, sem_a)

    def pair(t, carry):
        j0 = 2 * t
        pltpu.make_async_copy(hproj_hbm.at[gidx_v.at[j0]], rows_a, sem_a).wait()
        pltpu.async_copy(hproj_hbm.at[gidx_v.at[j0 + 1]], rows_b, sem_b)
        pltpu.sync_copy(rows_a, acc_sh.at[dst_v.at[j0]], add=True)
        pltpu.make_async_copy(hproj_hbm.at[gidx_v.at[j0 + 1]], rows_b,
                              sem_b).wait()
        pltpu.async_copy(hproj_hbm.at[gidx_v.at[j0 + 2]], rows_a, sem_a)
        pltpu.sync_copy(rows_b, acc_sh.at[dst_v.at[j0 + 1]], add=True)
        return carry

    lax.fori_loop(0, (_NCHUNK - 1) // 2, pair, 0)
    pltpu.make_async_copy(hproj_hbm.at[gidx_v.at[_NCHUNK - 1]], rows_a,
                          sem_a).wait()
    pltpu.sync_copy(rows_a, acc_sh.at[dst_v.at[_NCHUNK - 1]], add=True)

    plsc.subcore_barrier()
    pltpu.sync_copy(acc_sh.at[pl.ds(s * _RPT, _RPT)],
                    out_hbm.at[c].at[pl.ds(s * _RPT, _RPT)])


def kernel(x, edge_index, etypes, W_in, b_in, basis0, coeff0, Wself0, bias0,
           basis1, coeff1, Wself1, bias1, W_out, b_out):
    src = edge_index[0]
    dst = edge_index[1]

    et2d = etypes.reshape(E // 128, 128)
    src2d = src.reshape(E // 128, 128)
    dst2d = dst.reshape(_NW * _NCHUNK, _CH)
    b_in2 = b_in.reshape(1, HID)
    bias0_2 = bias0.reshape(1, HID)
    bias1_2 = bias1.reshape(1, HID)
    b_out2 = b_out.reshape(1, D_OUT)
    basis0_2d = basis0.reshape(NUM_BASES, HID * HID)
    basis1_2d = basis1.reshape(NUM_BASES, HID * HID)
    zeros = jnp.zeros((_RPT, HID), jnp.float32)

    w0_2d, w1_2d, gidx = _prep(coeff0, basis0_2d, coeff1, basis1_2d, et2d,
                               src2d)
    W0 = w0_2d.reshape(NUM_RELS, HID, HID)
    W1 = w1_2d.reshape(NUM_RELS, HID, HID)
    gidx2d = gidx.reshape(_NW * _NCHUNK, _CH)

    hproj0, selfp0 = _proj_first(x, W_in, b_in2, W0, Wself0, bias0_2)
    aggp0 = _sc_edge_agg(hproj0.reshape(NUM_RELS * N, HID), gidx2d, dst2d,
                         zeros)
    hproj1, selfp1 = _proj_mid(aggp0, selfp0, W1, Wself1, bias1_2)
    aggp1 = _sc_edge_agg(hproj1.reshape(NUM_RELS * N, HID), gidx2d, dst2d,
                         zeros)
    return _out_proj(aggp1, selfp1, W_out, b_out2)


# speedup vs baseline: 19.7740x; 19.7740x over previous
import functools

import jax
import jax.numpy as jnp
from jax import lax
from jax.experimental import pallas as pl
from jax.experimental.pallas import tpu as pltpu
from jax.experimental.pallas import tpu_sc as plsc

N = 10000
E = 320000
D_IN = 128
HID = 128
D_OUT = 128
NUM_RELS = 8
NUM_BASES = 4

_NC = 2
_NS = 16
_NW = _NC * _NS
_ET = E // _NW
_CH = 125
_NCHUNK = _ET // _CH

_SPLIT = 9600
_NA = _SPLIT + 8
_NB = (N - _SPLIT) + 8

_BLK = 400
_NBLK = N // _BLK


def _prep_body(coeff0_ref, basis0_ref, coeff1_ref, basis1_ref, et_ref, src_ref,
               dst_ref, w0_ref, w1_ref, gidx_ref, dsta_ref, dstb_ref):
    w0_ref[...] = jnp.dot(coeff0_ref[...], basis0_ref[...],
                          preferred_element_type=jnp.float32)
    w1_ref[...] = jnp.dot(coeff1_ref[...], basis1_ref[...],
                          preferred_element_type=jnp.float32)
    gidx_ref[...] = et_ref[...] * N + src_ref[...]
    d = dst_ref[...]
    spread = jnp.bitwise_and(d, 7)
    dsta_ref[...] = jnp.where(d < _SPLIT, d, _SPLIT + spread)
    dstb_ref[...] = jnp.where(d >= _SPLIT, d - _SPLIT, (N - _SPLIT) + spread)


def _prep(coeff0, basis0_2d, coeff1, basis1_2d, et2d, src2d, dst2d):
    return pl.pallas_call(
        _prep_body,
        out_shape=(
            jax.ShapeDtypeStruct((NUM_RELS, HID * HID), jnp.float32),
            jax.ShapeDtypeStruct((NUM_RELS, HID * HID), jnp.float32),
            jax.ShapeDtypeStruct((E // 128, 128), jnp.int32),
            jax.ShapeDtypeStruct((E // 128, 128), jnp.int32),
            jax.ShapeDtypeStruct((E // 128, 128), jnp.int32),
        ),
    )(coeff0, basis0_2d, coeff1, basis1_2d, et2d, src2d, dst2d)


def _proj_first_body(x_ref, win_ref, bin_ref, w_ref, wself_ref, bias_ref,
                     hproj_ref, selfp_ref):
    h = jnp.maximum(
        jnp.dot(x_ref[...], win_ref[...], preferred_element_type=jnp.float32)
        + bin_ref[...], 0.0)
    selfp_ref[...] = jnp.dot(h, wself_ref[...],
                             preferred_element_type=jnp.float32) + bias_ref[...]
    for r in range(NUM_RELS):
        hproj_ref[r] = jnp.dot(h, w_ref[r], preferred_element_type=jnp.float32)


def _combine(agga_ref, aggb_ref, selfp_prev_ref):
    i = pl.program_id(0)
    low = agga_ref[0] + agga_ref[1]
    high = aggb_ref[0] + aggb_ref[1]
    comb = jnp.where(i < _NBLK - 1, low, high)
    return jnp.maximum(comb + selfp_prev_ref[...], 0.0)


def _proj_mid_body(agga_ref, aggb_ref, selfp_prev_ref, w_ref, wself_ref,
                   bias_ref, hproj_ref, selfp_ref):
    h = _combine(agga_ref, aggb_ref, selfp_prev_ref)
    selfp_ref[...] = jnp.dot(h, wself_ref[...],
                             preferred_element_type=jnp.float32) + bias_ref[...]
    for r in range(NUM_RELS):
        hproj_ref[r] = jnp.dot(h, w_ref[r], preferred_element_type=jnp.float32)


def _out_body(agga_ref, aggb_ref, selfp_prev_ref, wout_ref, bout_ref, out_ref):
    h = _combine(agga_ref, aggb_ref, selfp_prev_ref)
    out_ref[...] = jnp.maximum(
        jnp.dot(h, wout_ref[...], preferred_element_type=jnp.float32)
        + bout_ref[...], 0.0)


def _proj_first(x, W_in, b_in, W, Wself, bias):
    return pl.pallas_call(
        _proj_first_body,
        grid=(_NBLK,),
        in_specs=[
            pl.BlockSpec((_BLK, D_IN), lambda i: (i, 0)),
            pl.BlockSpec((D_IN, HID), lambda i: (0, 0)),
            pl.BlockSpec((1, HID), lambda i: (0, 0)),
            pl.BlockSpec((NUM_RELS, HID, HID), lambda i: (0, 0, 0)),
            pl.BlockSpec((HID, HID), lambda i: (0, 0)),
            pl.BlockSpec((1, HID), lambda i: (0, 0)),
        ],
        out_specs=[
            pl.BlockSpec((NUM_RELS, _BLK, HID), lambda i: (0, i, 0)),
            pl.BlockSpec((_BLK, HID), lambda i: (i, 0)),
        ],
        out_shape=(
            jax.ShapeDtypeStruct((NUM_RELS, N, HID), jnp.float32),
            jax.ShapeDtypeStruct((N, HID), jnp.float32),
        ),
    )(x, W_in, b_in, W, Wself, bias)


_agga_spec = pl.BlockSpec((_NC, _BLK, HID),
                          lambda i: (0, jnp.minimum(i, _NBLK - 2), 0))
_aggb_spec = pl.BlockSpec((_NC, N - _SPLIT, HID), lambda i: (0, 0, 0))


def _proj_mid(aggpa, aggpb, selfp_prev, W, Wself, bias):
    return pl.pallas_call(
        _proj_mid_body,
        grid=(_NBLK,),
        in_specs=[
            _agga_spec,
            _aggb_spec,
            pl.BlockSpec((_BLK, HID), lambda i: (i, 0)),
            pl.BlockSpec((NUM_RELS, HID, HID), lambda i: (0, 0, 0)),
            pl.BlockSpec((HID, HID), lambda i: (0, 0)),
            pl.BlockSpec((1, HID), lambda i: (0, 0)),
        ],
        out_specs=[
            pl.BlockSpec((NUM_RELS, _BLK, HID), lambda i: (0, i, 0)),
            pl.BlockSpec((_BLK, HID), lambda i: (i, 0)),
        ],
        out_shape=(
            jax.ShapeDtypeStruct((NUM_RELS, N, HID), jnp.float32),
            jax.ShapeDtypeStruct((N, HID), jnp.float32),
        ),
    )(aggpa, aggpb, selfp_prev, W, Wself, bias)


def _out_proj(aggpa, aggpb, selfp_prev, W_out, b_out):
    return pl.pallas_call(
        _out_body,
        grid=(_NBLK,),
        in_specs=[
            _agga_spec,
            _aggb_spec,
            pl.BlockSpec((_BLK, HID), lambda i: (i, 0)),
            pl.BlockSpec((HID, D_OUT), lambda i: (0, 0)),
            pl.BlockSpec((1, D_OUT), lambda i: (0, 0)),
        ],
        out_specs=pl.BlockSpec((_BLK, D_OUT), lambda i: (i, 0)),
        out_shape=jax.ShapeDtypeStruct((N, D_OUT), jnp.float32),
    )(aggpa, aggpb, selfp_prev, W_out, b_out)


_sc_mesh = plsc.VectorSubcoreMesh(core_axis_name="c", subcore_axis_name="s")


def _make_sc_agg(nacc):
    q = (nacc // (_NS * 8)) * 8
    last = nacc - (_NS - 1) * q

    def _zero_span(zeros_hbm, acc_sh, base, nrows):
        nfull, rem = divmod(nrows, 128)
        for k in range(nfull):
            pltpu.sync_copy(zeros_hbm, acc_sh.at[pl.ds(base + k * 128, 128)])
        if rem:
            pltpu.sync_copy(zeros_hbm.at[pl.ds(0, rem)],
                            acc_sh.at[pl.ds(base + nfull * 128, rem)])

    @functools.partial(
        pl.kernel,
        out_type=jax.ShapeDtypeStruct((_NC, nacc, HID), jnp.float32),
        mesh=_sc_mesh,
        scratch_types=[
            pltpu.VMEM((_NCHUNK, _CH), jnp.int32),
            pltpu.VMEM((_NCHUNK, _CH), jnp.int32),
            pltpu.VMEM((_CH, HID), jnp.float32),
            pltpu.VMEM((_CH, HID), jnp.float32),
            pltpu.VMEM_SHARED((nacc, HID), jnp.float32),
            pltpu.SemaphoreType.DMA,
            pltpu.SemaphoreType.DMA,
        ],
    )
    def _sc_agg(hproj_hbm, gidx_hbm, dst_hbm, zeros_hbm, out_hbm,
                gidx_v, dst_v, rows_a, rows_b, acc_sh, sem_a, sem_b):
        c = lax.axis_index("c")
        s = lax.axis_index("s")
        wid = c * _NS + s

        pltpu.sync_copy(gidx_hbm.at[pl.ds(wid * _NCHUNK, _NCHUNK)], gidx_v)
        pltpu.sync_copy(dst_hbm.at[pl.ds(wid * _NCHUNK, _NCHUNK)], dst_v)

        @pl.when(s < _NS - 1)
        def _():
            _zero_span(zeros_hbm, acc_sh, s * q, q)

        @pl.when(s == _NS - 1)
        def _():
            _zero_span(zeros_hbm, acc_sh, (_NS - 1) * q, last)

        plsc.subcore_barrier()

        pltpu.async_copy(hproj_hbm.at[gidx_v.at[0]], rows_a, sem_a)

        def pair(t, carry):
            j0 = 2 * t
            pltpu.make_async_copy(hproj_hbm.at[gidx_v.at[j0]], rows_a,
                                  sem_a).wait()
            pltpu.async_copy(hproj_hbm.at[gidx_v.at[j0 + 1]], rows_b, sem_b)
            pltpu.sync_copy(rows_a, acc_sh.at[dst_v.at[j0]], add=True)
            pltpu.make_async_copy(hproj_hbm.at[gidx_v.at[j0 + 1]], rows_b,
                                  sem_b).wait()

            @pl.when(j0 + 2 < _NCHUNK)
            def _():
                pltpu.async_copy(hproj_hbm.at[gidx_v.at[j0 + 2]], rows_a,
                                 sem_a)

            pltpu.sync_copy(rows_b, acc_sh.at[dst_v.at[j0 + 1]], add=True)
            return carry

        lax.fori_loop(0, _NCHUNK // 2, pair, 0)

        plsc.subcore_barrier()

        @pl.when(s < _NS - 1)
        def _():
            pltpu.sync_copy(acc_sh.at[pl.ds(s * q, q)],
                            out_hbm.at[c].at[pl.ds(s * q, q)])

        @pl.when(s == _NS - 1)
        def _():
            pltpu.sync_copy(acc_sh.at[pl.ds((_NS - 1) * q, last)],
                            out_hbm.at[c].at[pl.ds((_NS - 1) * q, last)])

    return _sc_agg


_sc_agg_low = _make_sc_agg(_NA)
_sc_agg_high = _make_sc_agg(_NB)


def kernel(x, edge_index, etypes, W_in, b_in, basis0, coeff0, Wself0, bias0,
           basis1, coeff1, Wself1, bias1, W_out, b_out):
    src = edge_index[0]
    dst = edge_index[1]

    et2d = etypes.reshape(E // 128, 128)
    src2d = src.reshape(E // 128, 128)
    dst2d = dst.reshape(E // 128, 128)
    b_in2 = b_in.reshape(1, HID)
    bias0_2 = bias0.reshape(1, HID)
    bias1_2 = bias1.reshape(1, HID)
    b_out2 = b_out.reshape(1, D_OUT)
    basis0_2d = basis0.reshape(NUM_BASES, HID * HID)
    basis1_2d = basis1.reshape(NUM_BASES, HID * HID)
    zeros = jnp.zeros((128, HID), jnp.float32)

    w0_2d, w1_2d, gidx, dsta, dstb = _prep(coeff0, basis0_2d, coeff1,
                                           basis1_2d, et2d, src2d, dst2d)
    W0 = w0_2d.reshape(NUM_RELS, HID, HID)
    W1 = w1_2d.reshape(NUM_RELS, HID, HID)
    gidx2d = gidx.reshape(_NW * _NCHUNK, _CH)
    dsta2d = dsta.reshape(_NW * _NCHUNK, _CH)
    dstb2d = dstb.reshape(_NW * _NCHUNK, _CH)

    hproj0, selfp0 = _proj_first(x, W_in, b_in2, W0, Wself0, bias0_2)
    h0_2d = hproj0.reshape(NUM_RELS * N, HID)
    aggpa0 = _sc_agg_low(h0_2d, gidx2d, dsta2d, zeros)
    aggpb0 = _sc_agg_high(h0_2d, gidx2d, dstb2d, zeros)
    hproj1, selfp1 = _proj_mid(aggpa0, aggpb0, selfp0, W1, Wself1, bias1_2)
    h1_2d = hproj1.reshape(NUM_RELS * N, HID)
    aggpa1 = _sc_agg_low(h1_2d, gidx2d, dsta2d, zeros)
    aggpb1 = _sc_agg_high(h1_2d, gidx2d, dstb2d, zeros)
    return _out_proj(aggpa1, aggpb1, selfp1, W_out, b_out2)
